# Initial kernel scaffold; baseline (speedup 1.0000x reference)
import jax
import jax.numpy as jnp
from jax.experimental import pallas as pl


def kernel(words, W, root):
    raise NotImplementedError("write your pallas kernel here")


# speedup vs baseline: 3.0848x; 3.0848x over previous
import jax
import jax.numpy as jnp
from jax.experimental import pallas as pl
from jax.experimental.pallas import tpu as pltpu

NT = 256
MG_U = 8


def _kernel(words_smem, w_hbm, wcol_ref, root_ref, out_ref,
            g0, g1, gt, tile, troot, dsem):
    b = pl.program_id(0)
    t = pl.program_id(1)
    ntiles = pl.num_programs(1)
    n = tile.shape[0]

    def issue_rows(g_ref, sem, tt):
        base = tt * NT

        def body(i, _):
            r = words_smem[b, base + i]
            pltpu.make_async_copy(w_hbm.at[r], g_ref.at[i], sem).start()
            return 0

        jax.lax.fori_loop(0, NT, body, 0)

    def wait_rows(g_ref, sem):
        pltpu.make_async_copy(w_hbm.at[pl.ds(0, NT)], g_ref, sem).wait()

    slot = jax.lax.rem(t, 2)

    @pl.when(t == 0)
    def _():
        issue_rows(g0, dsem.at[0], 0)

    @pl.when(slot == 0)
    def _():
        wait_rows(g0, dsem.at[0])

    @pl.when(slot == 1)
    def _():
        wait_rows(g1, dsem.at[1])

    @pl.when(jnp.logical_and(t + 1 < ntiles, slot == 0))
    def _():
        issue_rows(g1, dsem.at[1], t + 1)

    @pl.when(jnp.logical_and(t + 1 < ntiles, slot == 1))
    def _():
        issue_rows(g0, dsem.at[0], t + 1)

    @pl.when(slot == 0)
    def _():
        gt[:, 0, :] = g0[:, 0, :].T

    @pl.when(slot == 1)
    def _():
        gt[:, 0, :] = g1[:, 0, :].T

    def mg_body(mo, _):
        m0 = mo * MG_U
        for u in range(MG_U):
            c = words_smem[b, m0 + u]
            tile[m0 + u, 0] = gt[c, 0]
        return 0

    jax.lax.fori_loop(0, n // MG_U, mg_body, 0)

    def rg_body(io, _):
        for u in range(MG_U):
            i = io * MG_U + u
            q = words_smem[b, t * NT + i] >> 7
            troot[i, 0] = root_ref[q, 0]
        return 0

    jax.lax.fori_loop(0, NT // MG_U, rg_body, 0)

    o_blk = tile[:, 0, :].T
    rvals = wcol_ref[0, 0]
    lane = jax.lax.broadcasted_iota(jnp.int32, (NT, 128), 1)
    tr = troot[:, 0, :]
    rv = jnp.sum(jnp.where(lane == (rvals & 127), tr, 0.0),
                 axis=1, keepdims=True)
    row = jax.lax.broadcasted_iota(jnp.int32, (NT, n), 0)
    col = jax.lax.broadcasted_iota(jnp.int32, (NT, n), 1)
    diag = col == t * NT + row
    out_ref[0] = o_blk + jnp.where(diag, jnp.broadcast_to(rv, (NT, n)), 0.0)


def kernel(words, W, root):
    B, N = words.shape
    V = W.shape[0]
    ntiles = N // NT
    words = words.astype(jnp.int32)
    w3 = W.reshape(V, 1, V)
    vpad = ((V + 127) // 128) * 128
    rootp = jnp.pad(root, (0, vpad - V)).reshape(vpad // 128, 1, 128)
    wcol4 = words.reshape(B, ntiles, NT, 1)

    grid_spec = pltpu.PrefetchScalarGridSpec(
        num_scalar_prefetch=1,
        grid=(B, ntiles),
        in_specs=[
            pl.BlockSpec(memory_space=pl.ANY),
            pl.BlockSpec((1, 1, NT, 1), lambda b, t, w: (b, t, 0, 0)),
            pl.BlockSpec((vpad // 128, 1, 128), lambda b, t, w: (0, 0, 0)),
        ],
        out_specs=pl.BlockSpec((1, NT, N), lambda b, t, w: (b, t, 0)),
        scratch_shapes=[
            pltpu.VMEM((NT, 1, V), jnp.float32),
            pltpu.VMEM((NT, 1, V), jnp.float32),
            pltpu.VMEM((V, 1, NT), jnp.float32),
            pltpu.VMEM((N, 1, NT), jnp.float32),
            pltpu.VMEM((NT, 1, 128), jnp.float32),
            pltpu.SemaphoreType.DMA((2,)),
        ],
    )
    return pl.pallas_call(
        _kernel,
        out_shape=jax.ShapeDtypeStruct((B, N, N), jnp.float32),
        grid_spec=grid_spec,
        compiler_params=pltpu.CompilerParams(
            dimension_semantics=("parallel", "arbitrary"),
            vmem_limit_bytes=56 * 1024 * 1024,
        ),
        name="gather_bilinear",
    )(words, w3, wcol4, rootp)


# speedup vs baseline: 3.9577x; 1.2830x over previous
import jax
import jax.numpy as jnp
from jax.experimental import pallas as pl
from jax.experimental.pallas import tpu as pltpu

NT = 256
MG_U = 8


def _kernel(words_smem, w_hbm, wcol_ref, root_ref, out_ref,
            g0, g1, gt, tile, troot, dsem):
    b = pl.program_id(0)
    t = pl.program_id(1)
    ntiles = pl.num_programs(1)
    n = tile.shape[0]

    def issue_rows(g_ref, sem, tt):
        base = tt * NT

        def body(i, _):
            r = words_smem[b, base + i]
            pltpu.make_async_copy(w_hbm.at[pl.ds(r, 1), :], g_ref.at[i], sem).start()
            return 0

        jax.lax.fori_loop(0, NT, body, 0)

    def wait_rows(g_ref, sem):
        pltpu.make_async_copy(g_ref, g_ref, sem).wait()

    slot = jax.lax.rem(t, 2)

    @pl.when(t == 0)
    def _():
        issue_rows(g0, dsem.at[0], 0)

    @pl.when(slot == 0)
    def _():
        wait_rows(g0, dsem.at[0])

    @pl.when(slot == 1)
    def _():
        wait_rows(g1, dsem.at[1])

    @pl.when(jnp.logical_and(t + 1 < ntiles, slot == 0))
    def _():
        issue_rows(g1, dsem.at[1], t + 1)

    @pl.when(jnp.logical_and(t + 1 < ntiles, slot == 1))
    def _():
        issue_rows(g0, dsem.at[0], t + 1)

    @pl.when(slot == 0)
    def _():
        gt[:, 0, :] = g0[:, 0, :].T

    @pl.when(slot == 1)
    def _():
        gt[:, 0, :] = g1[:, 0, :].T

    def mg_body(mo, _):
        m0 = mo * MG_U
        for u in range(MG_U):
            c = words_smem[b, m0 + u]
            tile[m0 + u, 0] = gt[c, 0]
        return 0

    jax.lax.fori_loop(0, n // MG_U, mg_body, 0)

    def rg_body(io, _):
        for u in range(MG_U):
            i = io * MG_U + u
            q = words_smem[b, t * NT + i] >> 7
            troot[i, 0] = root_ref[q, 0]
        return 0

    jax.lax.fori_loop(0, NT // MG_U, rg_body, 0)

    o_blk = tile[:, 0, :].T
    rvals = wcol_ref[0, 0]
    lane = jax.lax.broadcasted_iota(jnp.int32, (NT, 128), 1)
    tr = troot[:, 0, :]
    rv = jnp.sum(jnp.where(lane == (rvals & 127), tr, 0.0),
                 axis=1, keepdims=True)
    row = jax.lax.broadcasted_iota(jnp.int32, (NT, n), 0)
    col = jax.lax.broadcasted_iota(jnp.int32, (NT, n), 1)
    diag = col == t * NT + row
    out_ref[0] = o_blk + jnp.where(diag, jnp.broadcast_to(rv, (NT, n)), 0.0)


def kernel(words, W, root):
    B, N = words.shape
    V = W.shape[0]
    ntiles = N // NT
    words = words.astype(jnp.int32)
    vpad = ((V + 127) // 128) * 128
    rootp = jnp.pad(root, (0, vpad - V)).reshape(vpad // 128, 1, 128)
    wcol4 = words.reshape(B, ntiles, NT, 1)

    grid_spec = pltpu.PrefetchScalarGridSpec(
        num_scalar_prefetch=1,
        grid=(B, ntiles),
        in_specs=[
            pl.BlockSpec(memory_space=pl.ANY),
            pl.BlockSpec((1, 1, NT, 1), lambda b, t, w: (b, t, 0, 0)),
            pl.BlockSpec((vpad // 128, 1, 128), lambda b, t, w: (0, 0, 0)),
        ],
        out_specs=pl.BlockSpec((1, NT, N), lambda b, t, w: (b, t, 0)),
        scratch_shapes=[
            pltpu.VMEM((NT, 1, V), jnp.float32),
            pltpu.VMEM((NT, 1, V), jnp.float32),
            pltpu.VMEM((V, 1, NT), jnp.float32),
            pltpu.VMEM((N, 1, NT), jnp.float32),
            pltpu.VMEM((NT, 1, 128), jnp.float32),
            pltpu.SemaphoreType.DMA((2,)),
        ],
    )
    return pl.pallas_call(
        _kernel,
        out_shape=jax.ShapeDtypeStruct((B, N, N), jnp.float32),
        grid_spec=grid_spec,
        compiler_params=pltpu.CompilerParams(
            dimension_semantics=("parallel", "arbitrary"),
            vmem_limit_bytes=56 * 1024 * 1024,
        ),
        name="gather_bilinear",
    )(words, W, wcol4, rootp)
